# Initial kernel scaffold; baseline (speedup 1.0000x reference)
import jax
import jax.numpy as jnp
from jax.experimental import pallas as pl


def kernel(x, table):
    raise NotImplementedError("write your pallas kernel here")


# speedup vs baseline: 1.2153x; 1.2153x over previous
import functools
import math

import numpy as np
import jax
import jax.numpy as jnp
from jax import lax
from jax.experimental import pallas as pl
from jax.experimental.pallas import tpu as pltpu
from jax.experimental.pallas import tpu_sc as plsc

D_MODEL = 768
LANES = 16
NUM_CORES = 2
NUM_SUBCORES = 16
NUM_WORKERS = NUM_CORES * NUM_SUBCORES
CHUNK_ROWS = 64
SCALE = float(np.float32(math.sqrt(D_MODEL)))


def _emb_body(table_hbm, idx_hbm, out_hbm, idx_v, rows_v, sem):
    num_chunks = idx_hbm.shape[1]
    wid = lax.axis_index("s") * NUM_CORES + lax.axis_index("c")
    base = wid * (num_chunks * CHUNK_ROWS)
    pltpu.sync_copy(idx_hbm.at[wid], idx_v)

    @pl.loop(0, num_chunks)
    def _(c):
        pltpu.async_copy(table_hbm.at[idx_v.at[c]], rows_v, sem).wait()

        @pl.loop(0, CHUNK_ROWS)
        def _(i):
            row = rows_v.at[i]
            for j in range(D_MODEL // LANES):
                sl = pl.ds(j * LANES, LANES)
                row[sl] = row[sl] * SCALE

        pltpu.sync_copy(rows_v, out_hbm.at[pl.ds(base + c * CHUNK_ROWS, CHUNK_ROWS)])


def kernel(x, table):
    batch = x.size
    rows_per_worker = batch // NUM_WORKERS
    num_chunks = rows_per_worker // CHUNK_ROWS
    idx = x.reshape(NUM_WORKERS, num_chunks, CHUNK_ROWS).astype(jnp.int32)

    mesh = plsc.VectorSubcoreMesh(core_axis_name="c", subcore_axis_name="s")
    k = functools.partial(
        pl.kernel,
        out_type=jax.ShapeDtypeStruct((batch, D_MODEL), jnp.float32),
        mesh=mesh,
        scratch_types=[
            pltpu.VMEM((num_chunks, CHUNK_ROWS), jnp.int32),
            pltpu.VMEM((CHUNK_ROWS, D_MODEL), jnp.float32),
            pltpu.SemaphoreType.DMA,
        ],
    )(_emb_body)
    out = k(table, idx)
    return out.reshape(*x.shape, D_MODEL)


# speedup vs baseline: 1.5629x; 1.2860x over previous
import functools
import math

import numpy as np
import jax
import jax.numpy as jnp
from jax import lax
from jax.experimental import pallas as pl
from jax.experimental.pallas import tpu as pltpu
from jax.experimental.pallas import tpu_sc as plsc

D_MODEL = 768
LANES = 16
NUM_CORES = 2
NUM_SUBCORES = 16
NUM_WORKERS = NUM_CORES * NUM_SUBCORES
CHUNK_ROWS = 64
SCALE = float(np.float32(math.sqrt(D_MODEL)))


def _scale_rows(buf):
    @pl.loop(0, CHUNK_ROWS)
    def _(i):
        row = buf.at[i]
        for j in range(D_MODEL // LANES):
            sl = pl.ds(j * LANES, LANES)
            row[sl] = row[sl] * SCALE


def _emb_body(table_hbm, idx_hbm, out_hbm, idx_v, rows0, rows1, g0, g1, o0, o1):
    num_chunks = idx_hbm.shape[1]
    wid = lax.axis_index("s") * NUM_CORES + lax.axis_index("c")
    base = wid * (num_chunks * CHUNK_ROWS)
    bufs, gsems, osems = (rows0, rows1), (g0, g1), (o0, o1)
    pltpu.sync_copy(idx_hbm.at[wid], idx_v)

    def gather(c):
        return table_hbm.at[idx_v.at[c]]

    def oslice(c):
        return out_hbm.at[pl.ds(base + c * CHUNK_ROWS, CHUNK_ROWS)]

    pltpu.async_copy(gather(0), bufs[0], gsems[0])
    if num_chunks > 1:
        pltpu.async_copy(gather(1), bufs[1], gsems[1])
    for c in range(num_chunks):
        b = c & 1
        pltpu.make_async_copy(gather(c), bufs[b], gsems[b]).wait()
        _scale_rows(bufs[b])
        pltpu.async_copy(bufs[b], oslice(c), osems[b])
        if c + 2 < num_chunks:
            pltpu.make_async_copy(bufs[b], oslice(c), osems[b]).wait()
            pltpu.async_copy(gather(c + 2), bufs[b], gsems[b])
    for c in range(max(num_chunks - 2, 0), num_chunks):
        b = c & 1
        pltpu.make_async_copy(bufs[b], oslice(c), osems[b]).wait()


def kernel(x, table):
    batch = x.size
    rows_per_worker = batch // NUM_WORKERS
    num_chunks = rows_per_worker // CHUNK_ROWS
    idx = x.reshape(NUM_WORKERS, num_chunks, CHUNK_ROWS).astype(jnp.int32)

    mesh = plsc.VectorSubcoreMesh(core_axis_name="c", subcore_axis_name="s")
    k = functools.partial(
        pl.kernel,
        out_type=jax.ShapeDtypeStruct((batch, D_MODEL), jnp.float32),
        mesh=mesh,
        scratch_types=[
            pltpu.VMEM((num_chunks, CHUNK_ROWS), jnp.int32),
            pltpu.VMEM((CHUNK_ROWS, D_MODEL), jnp.float32),
            pltpu.VMEM((CHUNK_ROWS, D_MODEL), jnp.float32),
            pltpu.SemaphoreType.DMA,
            pltpu.SemaphoreType.DMA,
            pltpu.SemaphoreType.DMA,
            pltpu.SemaphoreType.DMA,
        ],
    )(_emb_body)
    out = k(table, idx)
    return out.reshape(*x.shape, D_MODEL)
